# baseline (speedup 1.0000x reference)
import jax
from jax import lax
import jax.numpy as jnp
from jax.experimental import pallas as pl
from jax.experimental.pallas import tpu as pltpu
from jax.experimental.pallas import tpu_sc as plsc

DIM = 64
PAD = 128
NW = 32
NS = 16
CH = 128
NB = 8192
BB = 2048


def _transpose_pad(table):
    V = table.shape[0]
    tab_t = jnp.transpose(table)

    def body(t_ref, o_ref):
        o_ref[:, :DIM] = jnp.transpose(t_ref[...])

    return pl.pallas_call(
        body,
        grid=(pl.cdiv(V, NB),),
        in_specs=[pl.BlockSpec((DIM, NB), lambda i: (0, i))],
        out_specs=pl.BlockSpec((NB, PAD), lambda i: (i, 0)),
        out_shape=jax.ShapeDtypeStruct((V, PAD), jnp.float32),
    )(tab_t)


def _gather_jm(tab128, idx_cm, B, L):
    n = B * L
    rwl = n // NW
    C = rwl // CH
    cpp = B // CH

    mesh = plsc.VectorSubcoreMesh(core_axis_name="c", subcore_axis_name="s")

    @pl.kernel(
        out_type=jax.ShapeDtypeStruct((L, B, DIM), jnp.float32),
        mesh=mesh,
        scratch_types=[
            pltpu.VMEM((rwl,), jnp.int32),
            pltpu.VMEM((CH, PAD), jnp.float32),
            pltpu.VMEM((CH, PAD), jnp.float32),
            pltpu.VMEM((CH, DIM), jnp.float32),
            pltpu.VMEM((CH, DIM), jnp.float32),
            pltpu.SemaphoreType.DMA,
            pltpu.SemaphoreType.DMA,
            pltpu.SemaphoreType.DMA,
            pltpu.SemaphoreType.DMA,
        ],
    )
    def gather_kernel(tab_hbm, idx_hbm, out_hbm,
                      idx_v, g0, g1, o0, o1, gs0, gs1, ws0, ws1):
        wid = lax.axis_index("c") * NS + lax.axis_index("s")
        ibase = wid * rwl
        t0 = wid * C
        pltpu.sync_copy(idx_hbm.at[pl.ds(ibase, rwl)], idx_v)

        def gsrc(c):
            return tab_hbm.at[idx_v.at[pl.ds(c * CH, CH)]]

        def wdst(c):
            t = t0 + c
            return out_hbm.at[t // cpp, pl.ds((t % cpp) * CH, CH)]

        def compact(gbuf, obuf):
            @pl.loop(0, CH)
            def _(r):
                for k in range(DIM // 16):
                    obuf[r, pl.ds(k * 16, 16)] = gbuf[r, pl.ds(k * 16, 16)]

        pltpu.async_copy(gsrc(0), g0, gs0)
        pltpu.async_copy(gsrc(1), g1, gs1)

        def stage(c, gbuf, gsem, obuf, wsem):
            pltpu.make_async_copy(gsrc(c), gbuf, gsem).wait()

            @pl.when(c >= 2)
            def _():
                pltpu.make_async_copy(obuf, wdst(c - 2), wsem).wait()

            compact(gbuf, obuf)
            pltpu.async_copy(obuf, wdst(c), wsem)

            @pl.when(c + 2 < C)
            def _():
                pltpu.async_copy(gsrc(c + 2), gbuf, gsem)

        @pl.loop(0, C, step=2)
        def _(c):
            stage(c, g0, gs0, o0, ws0)
            stage(c + 1, g1, gs1, o1, ws1)

        pltpu.make_async_copy(o0, wdst(C - 2), ws0).wait()
        pltpu.make_async_copy(o1, wdst(C - 1), ws1).wait()

    return gather_kernel(tab128, idx_cm)


def _out_transpose(out_jm, B, L):
    def body(t_ref, o_ref):
        o_ref[0] = jnp.transpose(t_ref[0])

    return pl.pallas_call(
        body,
        grid=(L, B // BB),
        in_specs=[pl.BlockSpec((1, BB, DIM), lambda j, m: (j, m, 0))],
        out_specs=pl.BlockSpec((1, DIM, BB), lambda j, m: (j, 0, m)),
        out_shape=jax.ShapeDtypeStruct((L, DIM, B), jnp.float32),
    )(out_jm)


def kernel(head, tail, table):
    B, L = head.shape
    n = B * L
    head_i = jnp.transpose(head).reshape(n).astype(jnp.int32)
    tail_i = jnp.transpose(tail).reshape(n).astype(jnp.int32)
    tab128 = _transpose_pad(table)
    h_jm = _gather_jm(tab128, head_i, B, L)
    t_jm = _gather_jm(tab128, tail_i, B, L)
    h_t = _out_transpose(h_jm, B, L)
    t_t = _out_transpose(t_jm, B, L)
    return jnp.transpose(h_t, (2, 0, 1)), jnp.transpose(t_t, (2, 0, 1))


# speedup vs baseline: 1.1685x; 1.1685x over previous
import jax
from jax import lax
import jax.numpy as jnp
from jax.experimental import pallas as pl
from jax.experimental.pallas import tpu as pltpu
from jax.experimental.pallas import tpu_sc as plsc

DIM = 64
PAD = 128
NW = 32
NS = 16
CH = 128
NB = 16384
BB = 8192


def _transpose_pad(table):
    V = table.shape[0]
    tab_t = jnp.transpose(table)

    def body(t_ref, o_ref):
        o_ref[:, :DIM] = jnp.transpose(t_ref[...])

    return pl.pallas_call(
        body,
        grid=(pl.cdiv(V, NB),),
        in_specs=[pl.BlockSpec((DIM, NB), lambda i: (0, i))],
        out_specs=pl.BlockSpec((NB, PAD), lambda i: (i, 0)),
        out_shape=jax.ShapeDtypeStruct((V, PAD), jnp.float32),
    )(tab_t)


def _gather_jm(tab128, idx_cm, B, L):
    n = B * L
    rwl = n // NW
    C = rwl // CH
    cpp = B // CH

    mesh = plsc.VectorSubcoreMesh(core_axis_name="c", subcore_axis_name="s")

    @pl.kernel(
        out_type=jax.ShapeDtypeStruct((L, B, DIM), jnp.float32),
        mesh=mesh,
        scratch_types=[
            pltpu.VMEM((rwl,), jnp.int32),
            pltpu.VMEM((CH, PAD), jnp.float32),
            pltpu.VMEM((CH, PAD), jnp.float32),
            pltpu.VMEM((CH, DIM), jnp.float32),
            pltpu.VMEM((CH, DIM), jnp.float32),
            pltpu.SemaphoreType.DMA,
            pltpu.SemaphoreType.DMA,
            pltpu.SemaphoreType.DMA,
            pltpu.SemaphoreType.DMA,
        ],
    )
    def gather_kernel(tab_hbm, idx_hbm, out_hbm,
                      idx_v, g0, g1, o0, o1, gs0, gs1, ws0, ws1):
        wid = lax.axis_index("c") * NS + lax.axis_index("s")
        ibase = wid * rwl
        t0 = wid * C
        pltpu.sync_copy(idx_hbm.at[pl.ds(ibase, rwl)], idx_v)

        def gsrc(c):
            return tab_hbm.at[idx_v.at[pl.ds(c * CH, CH)]]

        def wdst(c):
            t = t0 + c
            return out_hbm.at[t // cpp, pl.ds((t % cpp) * CH, CH)]

        def compact(gbuf, obuf):
            @pl.loop(0, CH)
            def _(r):
                for k in range(DIM // 16):
                    obuf[r, pl.ds(k * 16, 16)] = gbuf[r, pl.ds(k * 16, 16)]

        pltpu.async_copy(gsrc(0), g0, gs0)
        pltpu.async_copy(gsrc(1), g1, gs1)

        def stage(c, gbuf, gsem, obuf, wsem):
            pltpu.make_async_copy(gsrc(c), gbuf, gsem).wait()

            @pl.when(c >= 2)
            def _():
                pltpu.make_async_copy(obuf, wdst(c - 2), wsem).wait()

            compact(gbuf, obuf)
            pltpu.async_copy(obuf, wdst(c), wsem)

            @pl.when(c + 2 < C)
            def _():
                pltpu.async_copy(gsrc(c + 2), gbuf, gsem)

        @pl.loop(0, C, step=2)
        def _(c):
            stage(c, g0, gs0, o0, ws0)
            stage(c + 1, g1, gs1, o1, ws1)

        pltpu.make_async_copy(o0, wdst(C - 2), ws0).wait()
        pltpu.make_async_copy(o1, wdst(C - 1), ws1).wait()

    return gather_kernel(tab128, idx_cm)


def _out_transpose(out_jm, B, L):
    def body(t_ref, o_ref):
        o_ref[0] = jnp.transpose(t_ref[0])

    return pl.pallas_call(
        body,
        grid=(L, B // BB),
        in_specs=[pl.BlockSpec((1, BB, DIM), lambda j, m: (j, m, 0))],
        out_specs=pl.BlockSpec((1, DIM, BB), lambda j, m: (j, 0, m)),
        out_shape=jax.ShapeDtypeStruct((L, DIM, B), jnp.float32),
    )(out_jm)


def kernel(head, tail, table):
    B, L = head.shape
    n = B * L
    head_i = jnp.transpose(head).reshape(n).astype(jnp.int32)
    tail_i = jnp.transpose(tail).reshape(n).astype(jnp.int32)
    tab128 = _transpose_pad(table)
    h_jm = _gather_jm(tab128, head_i, B, L)
    t_jm = _gather_jm(tab128, tail_i, B, L)
    h_t = _out_transpose(h_jm, B, L)
    t_t = _out_transpose(t_jm, B, L)
    return jnp.transpose(h_t, (2, 0, 1)), jnp.transpose(t_t, (2, 0, 1))
